# baseline (speedup 1.0000x reference)
import jax
import jax.numpy as jnp
from jax import lax
from jax.experimental import pallas as pl
from jax.experimental.pallas import tpu as pltpu
from jax.experimental.pallas import tpu_sc as plsc

B = 16384
FEAT = 64
DENSE_IN = 324
OUT = 128

NC = 2
NS = 16
NW = NC * NS
ROWS_PER_W = B // NW


def _sc_gather_body(a_idx, d_idx, p_idx, a_tab, d_tab, p_tab,
                    a_out, d_out, p_out,
                    idx_v, rows_v, sem_a, sem_d, sem_p):
    wid = lax.axis_index("s") * NC + lax.axis_index("c")
    base = wid * ROWS_PER_W

    pltpu.sync_copy(a_idx.at[wid], idx_v.at[0])
    pltpu.sync_copy(d_idx.at[wid], idx_v.at[1])
    pltpu.sync_copy(p_idx.at[wid], idx_v.at[2])

    sems = (sem_a, sem_d, sem_p)
    tabs = (a_tab, d_tab, p_tab)
    outs = (a_out, d_out, p_out)
    for t in range(3):
        def body(g, _, t=t):
            vec = idx_v[t, g // 4, pl.ds((g % 4) * 16, 16)]
            for l in range(16):
                i = vec[l]
                pltpu.async_copy(
                    tabs[t].at[i // 8, pl.ds(lax.rem(i, 8), 1)],
                    rows_v.at[pl.ds(g * 16 + l, 1)],
                    sems[t])
            return 0
        lax.fori_loop(0, ROWS_PER_W // 16, body, 0)
        pltpu.make_async_copy(
            outs[t].at[pl.ds(base, ROWS_PER_W)], rows_v, sems[t]).wait()
        pltpu.sync_copy(rows_v, outs[t].at[pl.ds(base, ROWS_PER_W)])


def _sc_gather(a_idx, d_idx, p_idx, a_tab, d_tab, p_tab):
    mesh = plsc.VectorSubcoreMesh(core_axis_name="c", subcore_axis_name="s",
                                  num_cores=NC, num_subcores=NS)
    f = pl.kernel(
        _sc_gather_body,
        out_type=(
            jax.ShapeDtypeStruct((B, FEAT), jnp.float32),
            jax.ShapeDtypeStruct((B, FEAT), jnp.float32),
            jax.ShapeDtypeStruct((B, FEAT), jnp.float32),
        ),
        mesh=mesh,
        scratch_types=[
            pltpu.VMEM((3, 8, 64), jnp.int32),
            pltpu.VMEM((ROWS_PER_W, FEAT), jnp.float32),
            pltpu.SemaphoreType.DMA,
            pltpu.SemaphoreType.DMA,
            pltpu.SemaphoreType.DMA,
        ],
    )
    return f(a_idx, d_idx, p_idx, a_tab, d_tab, p_tab)


def _gelu(x):
    return 0.5 * x * (1.0 + lax.erf(x * 0.7071067811865476))


def _ln(x):
    m = jnp.mean(x, axis=-1, keepdims=True)
    v = jnp.mean((x - m) * (x - m), axis=-1, keepdims=True)
    return (x - m) * lax.rsqrt(v + 1e-5)


def _mlp_body(title_ref, a_ref, d_ref, p_ref, dW_ref, db_ref,
              W1_ref, b1_ref, W2_ref, b2_ref, W3_ref, b3_ref, out_ref):
    f32 = jnp.float32
    dense = _gelu(lax.dot_general(
        title_ref[...], dW_ref[...],
        dimension_numbers=(((0,), (0,)), ((), ())),
        preferred_element_type=f32) + db_ref[...])
    combined = jnp.concatenate(
        [a_ref[...], d_ref[...], p_ref[...], dense], axis=1)
    h = jnp.dot(combined, W1_ref[...], preferred_element_type=f32) + b1_ref[...]
    h = _gelu(_ln(h))
    h = jnp.dot(h, W2_ref[...], preferred_element_type=f32) + b2_ref[...]
    h = _gelu(_ln(h))
    h = jnp.dot(h, W3_ref[...], preferred_element_type=f32) + b3_ref[...]
    out_ref[...] = _gelu(h)


BLK = 4096


def _tc_mlp(title, a_emb, d_emb, p_emb, dW, db, W1, b1, W2, b2, W3, b3):
    grid = (B // BLK,)
    full = lambda shape: pl.BlockSpec(shape, lambda i: (0, 0))
    return pl.pallas_call(
        _mlp_body,
        grid=grid,
        in_specs=[
            pl.BlockSpec((DENSE_IN, BLK), lambda i: (0, i)),
            pl.BlockSpec((BLK, FEAT), lambda i: (i, 0)),
            pl.BlockSpec((BLK, FEAT), lambda i: (i, 0)),
            pl.BlockSpec((BLK, FEAT), lambda i: (i, 0)),
            full((DENSE_IN, FEAT)),
            full((1, FEAT)),
            full((4 * FEAT, 128)),
            full((1, 128)),
            full((128, 64)),
            full((1, 64)),
            full((64, OUT)),
            full((1, OUT)),
        ],
        out_specs=pl.BlockSpec((BLK, OUT), lambda i: (i, 0)),
        out_shape=jax.ShapeDtypeStruct((B, OUT), jnp.float32),
    )(title, a_emb, d_emb, p_emb, dW, db, W1, b1, W2, b2, W3, b3)


def kernel(book_ids, book_authors, book_dates, book_publishers,
           book_title_embeddings,
           id_table, author_table, date_table, publisher_table,
           dense_W, dense_b, W1, b1, W2, b2, W3, b3):
    del book_ids, id_table
    a_emb, d_emb, p_emb = _sc_gather(
        book_authors.reshape(NW, 8, 64),
        book_dates.reshape(NW, 8, 64),
        book_publishers.reshape(NW, 8, 64),
        author_table.reshape(-1, 8, FEAT),
        date_table.reshape(-1, 8, FEAT),
        publisher_table.reshape(-1, 8, FEAT))
    return _tc_mlp(book_title_embeddings.T, a_emb, d_emb, p_emb,
                   dense_W, dense_b.reshape(1, FEAT),
                   W1, b1.reshape(1, 128),
                   W2, b2.reshape(1, 64),
                   W3, b3.reshape(1, OUT))


# speedup vs baseline: 1.0040x; 1.0040x over previous
import jax
import jax.numpy as jnp
from jax import lax
from jax.experimental import pallas as pl
from jax.experimental.pallas import tpu as pltpu
from jax.experimental.pallas import tpu_sc as plsc

B = 16384
FEAT = 64
DENSE_IN = 324
OUT = 128

NC = 2
NS = 16
NW = NC * NS
ROWS_PER_W = B // NW


def _sc_gather_body(a_idx, d_idx, p_idx, a_tab, d_tab, p_tab,
                    a_out, d_out, p_out,
                    idx_v, rows_v, sem_a, sem_d, sem_p):
    wid = lax.axis_index("s") * NC + lax.axis_index("c")
    base = wid * ROWS_PER_W

    pltpu.sync_copy(a_idx.at[wid], idx_v.at[0])
    pltpu.sync_copy(d_idx.at[wid], idx_v.at[1])
    pltpu.sync_copy(p_idx.at[wid], idx_v.at[2])

    sems = (sem_a, sem_d, sem_p)
    tabs = (a_tab, d_tab, p_tab)
    outs = (a_out, d_out, p_out)
    for t in range(3):
        def body(g, _, t=t):
            vec = idx_v[t, g // 4, pl.ds((g % 4) * 16, 16)]
            for l in range(16):
                i = vec[l]
                pltpu.async_copy(
                    tabs[t].at[i // 8, pl.ds(lax.rem(i, 8), 1)],
                    rows_v.at[pl.ds(g * 16 + l, 1)],
                    sems[t])
            return 0
        lax.fori_loop(0, ROWS_PER_W // 16, body, 0)
        pltpu.make_async_copy(
            outs[t].at[pl.ds(base, ROWS_PER_W)], rows_v, sems[t]).wait()
        pltpu.sync_copy(rows_v, outs[t].at[pl.ds(base, ROWS_PER_W)])


def _sc_gather(a_idx, d_idx, p_idx, a_tab, d_tab, p_tab):
    mesh = plsc.VectorSubcoreMesh(core_axis_name="c", subcore_axis_name="s",
                                  num_cores=NC, num_subcores=NS)
    f = pl.kernel(
        _sc_gather_body,
        out_type=(
            jax.ShapeDtypeStruct((B, FEAT), jnp.float32),
            jax.ShapeDtypeStruct((B, FEAT), jnp.float32),
            jax.ShapeDtypeStruct((B, FEAT), jnp.float32),
        ),
        mesh=mesh,
        scratch_types=[
            pltpu.VMEM((3, 8, 64), jnp.int32),
            pltpu.VMEM((ROWS_PER_W, FEAT), jnp.float32),
            pltpu.SemaphoreType.DMA,
            pltpu.SemaphoreType.DMA,
            pltpu.SemaphoreType.DMA,
        ],
    )
    return f(a_idx, d_idx, p_idx, a_tab, d_tab, p_tab)


def _gelu(x):
    return 0.5 * x * (1.0 + lax.erf(x * 0.7071067811865476))


def _ln(x):
    m = jnp.mean(x, axis=-1, keepdims=True)
    v = jnp.mean((x - m) * (x - m), axis=-1, keepdims=True)
    return (x - m) * lax.rsqrt(v + 1e-5)


def _mlp_body(title_ref, a_ref, d_ref, p_ref, dW_ref, db_ref,
              W1_ref, b1_ref, W2_ref, b2_ref, W3_ref, b3_ref, out_ref):
    f32 = jnp.float32
    dense = _gelu(lax.dot_general(
        title_ref[...], dW_ref[...],
        dimension_numbers=(((0,), (0,)), ((), ())),
        preferred_element_type=f32) + db_ref[...])
    combined = jnp.concatenate(
        [a_ref[...], d_ref[...], p_ref[...], dense], axis=1)
    h = jnp.dot(combined, W1_ref[...], preferred_element_type=f32) + b1_ref[...]
    h = _gelu(_ln(h))
    h = jnp.dot(h, W2_ref[...], preferred_element_type=f32) + b2_ref[...]
    h = _gelu(_ln(h))
    h = jnp.dot(h, W3_ref[...], preferred_element_type=f32) + b3_ref[...]
    out_ref[...] = _gelu(h)


BLK = 2048


def _tc_mlp(title, a_emb, d_emb, p_emb, dW, db, W1, b1, W2, b2, W3, b3):
    grid = (B // BLK,)
    full = lambda shape: pl.BlockSpec(shape, lambda i: (0, 0))
    return pl.pallas_call(
        _mlp_body,
        grid=grid,
        in_specs=[
            pl.BlockSpec((DENSE_IN, BLK), lambda i: (0, i)),
            pl.BlockSpec((BLK, FEAT), lambda i: (i, 0)),
            pl.BlockSpec((BLK, FEAT), lambda i: (i, 0)),
            pl.BlockSpec((BLK, FEAT), lambda i: (i, 0)),
            full((DENSE_IN, FEAT)),
            full((1, FEAT)),
            full((4 * FEAT, 128)),
            full((1, 128)),
            full((128, 64)),
            full((1, 64)),
            full((64, OUT)),
            full((1, OUT)),
        ],
        out_specs=pl.BlockSpec((BLK, OUT), lambda i: (i, 0)),
        out_shape=jax.ShapeDtypeStruct((B, OUT), jnp.float32),
    )(title, a_emb, d_emb, p_emb, dW, db, W1, b1, W2, b2, W3, b3)


def kernel(book_ids, book_authors, book_dates, book_publishers,
           book_title_embeddings,
           id_table, author_table, date_table, publisher_table,
           dense_W, dense_b, W1, b1, W2, b2, W3, b3):
    del book_ids, id_table
    a_emb, d_emb, p_emb = _sc_gather(
        book_authors.reshape(NW, 8, 64),
        book_dates.reshape(NW, 8, 64),
        book_publishers.reshape(NW, 8, 64),
        author_table.reshape(-1, 8, FEAT),
        date_table.reshape(-1, 8, FEAT),
        publisher_table.reshape(-1, 8, FEAT))
    return _tc_mlp(book_title_embeddings.T, a_emb, d_emb, p_emb,
                   dense_W, dense_b.reshape(1, FEAT),
                   W1, b1.reshape(1, 128),
                   W2, b2.reshape(1, 64),
                   W3, b3.reshape(1, OUT))
